# baseline (speedup 1.0000x reference)
import jax
import jax.numpy as jnp
from jax import lax
from jax.experimental import pallas as pl
from jax.experimental.pallas import tpu as pltpu
from jax.experimental.pallas import tpu_sc as plsc

N = 10000
D = 128
E = 320000
G = 128
NP = 10240
NT = 16
CHUNK = 128
SLAB = 40
ROWS_PER_TILE = NP // NT
DEG_ROWS = NP // 128
NCHW = 80

_f32 = jnp.float32
_i32 = jnp.int32


def _mesh():
    return plsc.VectorSubcoreMesh(core_axis_name="c", subcore_axis_name="s")


def _zero_rows(ref, nrows):
    zero16 = jnp.zeros((16,), _f32)

    def body(i, carry):
        for k in range(8):
            ref[i, pl.ds(k * 16, 16)] = zero16
        return carry

    lax.fori_loop(0, nrows, body, 0)


def _tile_range(s):
    start = (s * 156 + jnp.minimum(s, 4)) * 128
    nch = 156 + jnp.where(s < 4, 1, 0)
    return start, nch


def _deg_body(e_hbm, rowid_hbm, deg_out, dstw_v, deg1_v, deg2_v,
              rowid_v, deg_sh):
    c = lax.axis_index("c")
    s = lax.axis_index("s")
    zero16 = jnp.zeros((16,), _f32)

    def zbody(i, carry):
        deg1_v[pl.ds(i * 16, 16)] = zero16
        return carry

    lax.fori_loop(0, NP // 16, zbody, 0)
    _zero_rows(deg2_v, DEG_ROWS)

    @pl.when(s < 10)
    def _():
        pltpu.sync_copy(deg2_v.at[pl.ds(s * 8, 8)], deg_sh.at[pl.ds(s * 8, 8)])

    plsc.subcore_barrier()

    start, nch = _tile_range(s)
    winb = start + (nch - NCHW) * CHUNK
    w = NCHW * CHUNK

    pltpu.sync_copy(e_hbm.at[c, 1, pl.ds(start, w)], dstw_v.at[pl.ds(0, w)])
    pltpu.sync_copy(e_hbm.at[c, 1, pl.ds(winb, w)], dstw_v.at[pl.ds(w, w)])

    pltpu.sync_copy(rowid_hbm, rowid_v)
    ones16 = jnp.ones((16,), _f32)

    def body(j, carry):
        for k in range(8):
            idx = dstw_v[pl.ds(j * CHUNK + k * 16, 16)]
            plsc.addupdate_scatter(deg1_v, [idx], ones16)
        return carry

    lax.fori_loop(0, NCHW, body, 0)

    def bodyb(m, carry):
        for k in range(8):
            idx = dstw_v[pl.ds(w + m * CHUNK + k * 16, 16)]
            plsc.addupdate_scatter(deg1_v, [idx], ones16)
        return carry

    lax.fori_loop(160 - nch, NCHW, bodyb, 0)

    def rbody(r, carry):
        for k in range(8):
            deg2_v[r, pl.ds(k * 16, 16)] = deg1_v[pl.ds(r * 128 + k * 16, 16)]
        return carry

    lax.fori_loop(0, DEG_ROWS, rbody, 0)
    pltpu.sync_copy(deg2_v, deg_sh.at[rowid_v], add=True)
    plsc.subcore_barrier()

    @pl.when(s < 10)
    def _():
        pltpu.sync_copy(deg_sh.at[pl.ds(s * 8, 8)],
                        deg_out.at[c, pl.ds(s * 8, 8)])


def _deg_call(e_all, rowid):
    return pl.kernel(
        _deg_body,
        out_type=jax.ShapeDtypeStruct((2, DEG_ROWS, 128), _f32),
        mesh=_mesh(),
        scratch_types=[
            pltpu.VMEM((2 * NCHW * CHUNK,), _i32),
            pltpu.VMEM((NP,), _f32),
            pltpu.VMEM((DEG_ROWS, 128), _f32),
            pltpu.VMEM((DEG_ROWS,), _i32),
            pltpu.VMEM_SHARED((DEG_ROWS, 128), _f32),
        ],
        compiler_params=pltpu.CompilerParams(needs_layout_passes=False),
    )(e_all, rowid)


def _scatter_body(g_hbm, e_hbm, acc_out, src_v, dst_v, row_a,
                  row_b, acc_sh, sem_a, sem_b):
    c = lax.axis_index("c")
    s = lax.axis_index("s")
    pltpu.sync_copy(g_hbm.at[pl.ds(c * NP + s * ROWS_PER_TILE,
                                   ROWS_PER_TILE)],
                    acc_sh.at[pl.ds(s * ROWS_PER_TILE, ROWS_PER_TILE)])
    plsc.subcore_barrier()

    start, nch = _tile_range(s)
    sw = SLAB * CHUNK

    def load_slab(off):
        pltpu.sync_copy(e_hbm.at[c, 0, pl.ds(off, sw)], src_v)
        pltpu.sync_copy(e_hbm.at[c, 1, pl.ds(off, sw)], dst_v)
        npv = jnp.zeros((16,), _i32) + c * NP

        def addnp(i, carry):
            src_v[pl.ds(i * 16, 16)] = src_v[pl.ds(i * 16, 16)] + npv
            return carry

        lax.fori_loop(0, sw // 16, addnp, 0)

    def sv(j):
        return src_v.at[pl.ds(j * CHUNK, CHUNK)]

    def dv(j):
        return dst_v.at[pl.ds(j * CHUNK, CHUNK)]

    def run_pairs(first, npairs):
        pltpu.async_copy(g_hbm.at[sv(first)], row_a, sem_a)

        def pair(i, carry):
            j = first + 2 * i
            pltpu.async_copy(g_hbm.at[sv(j + 1)], row_b, sem_b)
            pltpu.make_async_copy(g_hbm.at[sv(j)], row_a, sem_a).wait()
            pltpu.sync_copy(row_a, acc_sh.at[dv(j)], add=True)

            @pl.when(i < npairs - 1)
            def _():
                pltpu.async_copy(g_hbm.at[sv(j + 2)], row_a, sem_a)

            pltpu.make_async_copy(g_hbm.at[sv(j + 1)], row_b, sem_b).wait()
            pltpu.sync_copy(row_b, acc_sh.at[dv(j + 1)], add=True)
            return carry

        lax.fori_loop(0, npairs, pair, 0)

    for q in range(3):
        load_slab(start + q * sw)
        run_pairs(0, SLAB // 2)

    load_slab(start + (nch - SLAB) * CHUNK)

    @pl.when(s < 4)
    def _():
        pltpu.sync_copy(g_hbm.at[sv(3)], row_a)
        pltpu.sync_copy(row_a, acc_sh.at[dv(3)], add=True)

    run_pairs(4, (SLAB - 4) // 2)

    plsc.subcore_barrier()
    pltpu.sync_copy(acc_sh.at[pl.ds(s * ROWS_PER_TILE, ROWS_PER_TILE)],
                    acc_out.at[c, pl.ds(s * ROWS_PER_TILE, ROWS_PER_TILE)])


def _scatter_call(g_all, e_all):
    return pl.kernel(
        _scatter_body,
        out_type=jax.ShapeDtypeStruct((2, NP, D), _f32),
        mesh=_mesh(),
        scratch_types=[
            pltpu.VMEM((SLAB * CHUNK,), _i32),
            pltpu.VMEM((SLAB * CHUNK,), _i32),
            pltpu.VMEM((CHUNK, D), _f32),
            pltpu.VMEM((CHUNK, D), _f32),
            pltpu.VMEM_SHARED((NP, D), _f32),
            pltpu.SemaphoreType.DMA,
            pltpu.SemaphoreType.DMA,
        ],
        compiler_params=pltpu.CompilerParams(needs_layout_passes=False),
    )(g_all, e_all)


_BLK = 1024
_DROWS = _BLK // 128


def _to_col(img):
    eye = jnp.eye(128, dtype=_f32)
    parts = [lax.dot_general(eye, img[r:r + 1, :], (((1,), (1,)), ((), ())),
                             preferred_element_type=_f32)
             for r in range(img.shape[0])]
    return jnp.concatenate(parts, axis=0)


def _g_body(x_ref, deg_ref, w_ref, g_ref, dis_ref):
    j = pl.program_id(1)
    node = (lax.broadcasted_iota(_i32, (_DROWS, 128), 0) * 128
            + lax.broadcasted_iota(_i32, (_DROWS, 128), 1)) + j * _BLK
    dis_img = jnp.where(node < N, lax.rsqrt(deg_ref[0] + 1.0), 0.0)
    dis_ref[0] = dis_img
    dis_col = _to_col(dis_img)
    g_ref[0] = jnp.dot(x_ref[0] * dis_col, w_ref[0],
                       preferred_element_type=_f32)


def _g_call(x_all, deg2d, w_all):
    return pl.pallas_call(
        _g_body,
        grid=(2, NP // _BLK),
        in_specs=[
            pl.BlockSpec((1, _BLK, D), lambda c, j: (c, j, 0)),
            pl.BlockSpec((1, _DROWS, 128), lambda c, j: (c, j, 0)),
            pl.BlockSpec((1, D, D), lambda c, j: (c, 0, 0)),
        ],
        out_specs=[
            pl.BlockSpec((1, _BLK, D), lambda c, j: (c, j, 0)),
            pl.BlockSpec((1, _DROWS, 128), lambda c, j: (c, j, 0)),
        ],
        out_shape=[
            jax.ShapeDtypeStruct((2, NP, D), _f32),
            jax.ShapeDtypeStruct((2, DEG_ROWS, 128), _f32),
        ],
    )(x_all, deg2d, w_all)


def _pool_body(acc_ref, dis_ref, b_ref, batch_ref, fcw_ref, fcb_ref,
               fw_ref, fb_ref, out_ref, ss_ref, cnt_ref):
    c = pl.program_id(0)
    j = pl.program_id(1)
    dis_col = _to_col(dis_ref[0])
    batch_col = _to_col(batch_ref[0].astype(_f32))
    y = dis_col * acc_ref[0] + b_ref[0]
    y = jnp.where(y >= 0, y, 0.01 * y)
    gid = lax.broadcasted_iota(_i32, (1, G), 1).astype(_f32)
    oh = (batch_col == gid).astype(_f32)
    ss = lax.dot_general(oh, y, (((0,), (0,)), ((), ())),
                         preferred_element_type=_f32)
    cn = lax.dot_general(oh, jnp.ones((_BLK, 1), _f32),
                         (((0,), (0,)), ((), ())),
                         preferred_element_type=_f32)

    @pl.when(jnp.logical_and(c == 0, j == 0))
    def _():
        ss_ref[...] = jnp.zeros((2, G, D), _f32)
        cnt_ref[...] = jnp.zeros((2, G, 1), _f32)

    @pl.when(c == 0)
    def _():
        ss_ref[0] += ss
        cnt_ref[0] += cn

    @pl.when(c == 1)
    def _():
        ss_ref[1] += ss
        cnt_ref[1] += cn

    @pl.when(jnp.logical_and(c == 1, j == NP // _BLK - 1))
    def _():
        o = None
        for cidx in range(2):
            m = ss_ref[cidx] / jnp.maximum(cnt_ref[cidx], 1.0)
            z = jnp.dot(m, fcw_ref[cidx], preferred_element_type=_f32) \
                + fcb_ref[cidx]
            z = jnp.where(z >= 0, z, 0.01 * z)
            fw = fw_ref[pl.ds(cidx * D, D), :]
            contrib = jnp.dot(z, fw, preferred_element_type=_f32)
            o = contrib if o is None else o + contrib
        out_ref[...] = o + fb_ref[...]


def _pool_call(acc, dis_col, b_all, batch_all, fcw_all, fcb_all,
               final_w, final_b):
    full = lambda c, j: (0, 0)
    return pl.pallas_call(
        _pool_body,
        grid=(2, NP // _BLK),
        in_specs=[
            pl.BlockSpec((1, _BLK, D), lambda c, j: (c, j, 0)),
            pl.BlockSpec((1, _DROWS, 128), lambda c, j: (c, j, 0)),
            pl.BlockSpec((1, 1, D), lambda c, j: (c, 0, 0)),
            pl.BlockSpec((1, _DROWS, 128), lambda c, j: (c, j, 0)),
            pl.BlockSpec((2, D, D), lambda c, j: (0, 0, 0)),
            pl.BlockSpec((2, 1, D), lambda c, j: (0, 0, 0)),
            pl.BlockSpec((2 * D, 1), full),
            pl.BlockSpec((1, 1), full),
        ],
        out_specs=pl.BlockSpec((G, 1), full),
        out_shape=jax.ShapeDtypeStruct((G, 1), _f32),
        scratch_shapes=[
            pltpu.VMEM((2, G, D), _f32),
            pltpu.VMEM((2, G, 1), _f32),
        ],
    )(acc, dis_col, b_all, batch_all, fcw_all, fcb_all, final_w, final_b)


def kernel(pro1_x, pro1_edge_index, pro1_batch, pro2_x, pro2_edge_index,
           pro2_batch, W1, b1, fc1_w, fc1_b, W2, b2, fc2_w, fc2_b,
           final_w, final_b):
    e_all = jnp.stack([pro1_edge_index, pro2_edge_index])
    rowid = jnp.arange(DEG_ROWS, dtype=_i32)

    x_all = jnp.pad(jnp.stack([pro1_x, pro2_x]), ((0, 0), (0, NP - N), (0, 0)))
    w_all = jnp.stack([W1, W2])
    b_all = jnp.stack([b1, b2]).reshape(2, 1, D)
    batch_all = jnp.pad(jnp.stack([pro1_batch, pro2_batch]),
                        ((0, 0), (0, NP - N)),
                        constant_values=G).reshape(2, DEG_ROWS, 128)
    fcw_all = jnp.stack([fc1_w, fc2_w])
    fcb_all = jnp.stack([fc1_b, fc2_b]).reshape(2, 1, D)
    fb = final_b.reshape(1, 1)

    deg2d = _deg_call(e_all, rowid)
    g, dis_img = _g_call(x_all, deg2d, w_all)
    g_all = g.reshape(2 * NP, D)
    acc = _scatter_call(g_all, e_all)
    return _pool_call(acc, dis_img, b_all, batch_all, fcw_all, fcb_all,
                      final_w, fb)


# speedup vs baseline: 1.0066x; 1.0066x over previous
import jax
import jax.numpy as jnp
from jax import lax
from jax.experimental import pallas as pl
from jax.experimental.pallas import tpu as pltpu
from jax.experimental.pallas import tpu_sc as plsc

N = 10000
D = 128
E = 320000
G = 128
NP = 10240
NT = 16
CHUNK = 128
SLAB = 40
ROWS_PER_TILE = NP // NT
DEG_ROWS = NP // 128
NCHW = 80

_f32 = jnp.float32
_i32 = jnp.int32


def _mesh():
    return plsc.VectorSubcoreMesh(core_axis_name="c", subcore_axis_name="s")


def _zero_rows(ref, nrows):
    zero16 = jnp.zeros((16,), _f32)

    def body(i, carry):
        for k in range(8):
            ref[i, pl.ds(k * 16, 16)] = zero16
        return carry

    lax.fori_loop(0, nrows, body, 0)


def _tile_range(s):
    start = (s * 156 + jnp.minimum(s, 4)) * 128
    nch = 156 + jnp.where(s < 4, 1, 0)
    return start, nch


def _deg_body(e_hbm, rowid_hbm, deg_out, dstw_v, deg1_v, deg2_v,
              rowid_v, deg_sh):
    c = lax.axis_index("c")
    s = lax.axis_index("s")
    zero16 = jnp.zeros((16,), _f32)

    def zbody(i, carry):
        deg1_v[pl.ds(i * 16, 16)] = zero16
        return carry

    lax.fori_loop(0, NP // 16, zbody, 0)
    _zero_rows(deg2_v, DEG_ROWS)

    @pl.when(s < 10)
    def _():
        pltpu.sync_copy(deg2_v.at[pl.ds(s * 8, 8)], deg_sh.at[pl.ds(s * 8, 8)])

    plsc.subcore_barrier()

    start, nch = _tile_range(s)
    winb = start + (nch - NCHW) * CHUNK
    w = NCHW * CHUNK

    pltpu.sync_copy(e_hbm.at[c, 1, pl.ds(start, w)], dstw_v.at[pl.ds(0, w)])
    pltpu.sync_copy(e_hbm.at[c, 1, pl.ds(winb, w)], dstw_v.at[pl.ds(w, w)])

    pltpu.sync_copy(rowid_hbm, rowid_v)
    ones16 = jnp.ones((16,), _f32)

    def body(j, carry):
        for k in range(8):
            idx = dstw_v[pl.ds(j * CHUNK + k * 16, 16)]
            plsc.addupdate_scatter(deg1_v, [idx], ones16)
        return carry

    lax.fori_loop(0, NCHW, body, 0)

    def bodyb(m, carry):
        for k in range(8):
            idx = dstw_v[pl.ds(w + m * CHUNK + k * 16, 16)]
            plsc.addupdate_scatter(deg1_v, [idx], ones16)
        return carry

    lax.fori_loop(160 - nch, NCHW, bodyb, 0)

    def rbody(r, carry):
        for k in range(8):
            deg2_v[r, pl.ds(k * 16, 16)] = deg1_v[pl.ds(r * 128 + k * 16, 16)]
        return carry

    lax.fori_loop(0, DEG_ROWS, rbody, 0)
    pltpu.sync_copy(deg2_v, deg_sh.at[rowid_v], add=True)
    plsc.subcore_barrier()

    @pl.when(s < 10)
    def _():
        pltpu.sync_copy(deg_sh.at[pl.ds(s * 8, 8)],
                        deg_out.at[c, pl.ds(s * 8, 8)])


def _deg_call(e_all, rowid):
    return pl.kernel(
        _deg_body,
        out_type=jax.ShapeDtypeStruct((2, DEG_ROWS, 128), _f32),
        mesh=_mesh(),
        scratch_types=[
            pltpu.VMEM((2 * NCHW * CHUNK,), _i32),
            pltpu.VMEM((NP,), _f32),
            pltpu.VMEM((DEG_ROWS, 128), _f32),
            pltpu.VMEM((DEG_ROWS,), _i32),
            pltpu.VMEM_SHARED((DEG_ROWS, 128), _f32),
        ],
        compiler_params=pltpu.CompilerParams(needs_layout_passes=False),
    )(e_all, rowid)


def _scatter_body(g_hbm, e_hbm, acc_out, src_v, dst_v, row_a,
                  row_b, acc_sh, sem_a, sem_b):
    c = lax.axis_index("c")
    s = lax.axis_index("s")
    pltpu.sync_copy(g_hbm.at[pl.ds(c * NP + s * ROWS_PER_TILE,
                                   ROWS_PER_TILE)],
                    acc_sh.at[pl.ds(s * ROWS_PER_TILE, ROWS_PER_TILE)])
    plsc.subcore_barrier()

    start, nch = _tile_range(s)
    sw = SLAB * CHUNK

    def load_slab(off):
        pltpu.sync_copy(e_hbm.at[c, 0, pl.ds(off, sw)], src_v)
        pltpu.sync_copy(e_hbm.at[c, 1, pl.ds(off, sw)], dst_v)
        npv = jnp.zeros((16,), _i32) + c * NP

        def addnp(i, carry):
            src_v[pl.ds(i * 16, 16)] = src_v[pl.ds(i * 16, 16)] + npv
            return carry

        lax.fori_loop(0, sw // 16, addnp, 0)

    def sv(j):
        return src_v.at[pl.ds(j * CHUNK, CHUNK)]

    def dv(j):
        return dst_v.at[pl.ds(j * CHUNK, CHUNK)]

    def run_pairs(first, npairs):
        pltpu.async_copy(g_hbm.at[sv(first)], row_a, sem_a)

        def pair(i, carry):
            j = first + 2 * i
            pltpu.async_copy(g_hbm.at[sv(j + 1)], row_b, sem_b)
            pltpu.make_async_copy(g_hbm.at[sv(j)], row_a, sem_a).wait()
            pltpu.sync_copy(row_a, acc_sh.at[dv(j)], add=True)

            @pl.when(i < npairs - 1)
            def _():
                pltpu.async_copy(g_hbm.at[sv(j + 2)], row_a, sem_a)

            pltpu.make_async_copy(g_hbm.at[sv(j + 1)], row_b, sem_b).wait()
            pltpu.sync_copy(row_b, acc_sh.at[dv(j + 1)], add=True)
            return carry

        lax.fori_loop(0, npairs, pair, 0)

    for q in range(3):
        load_slab(start + q * sw)
        run_pairs(0, SLAB // 2)

    load_slab(start + (nch - SLAB) * CHUNK)

    @pl.when(s < 4)
    def _():
        pltpu.sync_copy(g_hbm.at[sv(3)], row_a)
        pltpu.sync_copy(row_a, acc_sh.at[dv(3)], add=True)

    run_pairs(4, (SLAB - 4) // 2)

    plsc.subcore_barrier()
    pltpu.sync_copy(acc_sh.at[pl.ds(s * ROWS_PER_TILE, ROWS_PER_TILE)],
                    acc_out.at[c, pl.ds(s * ROWS_PER_TILE, ROWS_PER_TILE)])


def _scatter_call(g_all, e_all):
    return pl.kernel(
        _scatter_body,
        out_type=jax.ShapeDtypeStruct((2, NP, D), _f32),
        mesh=_mesh(),
        scratch_types=[
            pltpu.VMEM((SLAB * CHUNK,), _i32),
            pltpu.VMEM((SLAB * CHUNK,), _i32),
            pltpu.VMEM((CHUNK, D), _f32),
            pltpu.VMEM((CHUNK, D), _f32),
            pltpu.VMEM_SHARED((NP, D), _f32),
            pltpu.SemaphoreType.DMA,
            pltpu.SemaphoreType.DMA,
        ],
        compiler_params=pltpu.CompilerParams(needs_layout_passes=False),
    )(g_all, e_all)


_BLK = 1024
_DROWS = _BLK // 128


def _to_col(img):
    eye = jnp.eye(128, dtype=_f32)
    parts = [lax.dot_general(eye, img[r:r + 1, :], (((1,), (1,)), ((), ())),
                             preferred_element_type=_f32)
             for r in range(img.shape[0])]
    return jnp.concatenate(parts, axis=0)


def _g_body(x_ref, deg_ref, w_ref, g_ref, dis_ref):
    j = pl.program_id(1)
    node = (lax.broadcasted_iota(_i32, (_DROWS, 128), 0) * 128
            + lax.broadcasted_iota(_i32, (_DROWS, 128), 1)) + j * _BLK
    dis_img = jnp.where(node < N, lax.rsqrt(deg_ref[0] + 1.0), 0.0)
    dis_ref[0] = dis_img
    dis_col = _to_col(dis_img)
    g_ref[0] = dis_col * jnp.dot(x_ref[0], w_ref[0],
                                 preferred_element_type=_f32)


def _g_call(x_all, deg2d, w_all):
    return pl.pallas_call(
        _g_body,
        grid=(2, NP // _BLK),
        in_specs=[
            pl.BlockSpec((1, _BLK, D), lambda c, j: (c, j, 0)),
            pl.BlockSpec((1, _DROWS, 128), lambda c, j: (c, j, 0)),
            pl.BlockSpec((1, D, D), lambda c, j: (c, 0, 0)),
        ],
        out_specs=[
            pl.BlockSpec((1, _BLK, D), lambda c, j: (c, j, 0)),
            pl.BlockSpec((1, _DROWS, 128), lambda c, j: (c, j, 0)),
        ],
        out_shape=[
            jax.ShapeDtypeStruct((2, NP, D), _f32),
            jax.ShapeDtypeStruct((2, DEG_ROWS, 128), _f32),
        ],
    )(x_all, deg2d, w_all)


def _pool_body(acc_ref, dis_ref, b_ref, batch_ref, fcw_ref, fcb_ref,
               fw_ref, fb_ref, out_ref, ss_ref, cnt_ref):
    c = pl.program_id(0)
    j = pl.program_id(1)
    dis_col = _to_col(dis_ref[0])
    batch_col = _to_col(batch_ref[0].astype(_f32))
    y = dis_col * acc_ref[0] + b_ref[0]
    y = jnp.where(y >= 0, y, 0.01 * y)
    gid = lax.broadcasted_iota(_i32, (1, G), 1).astype(_f32)
    oh = (batch_col == gid).astype(_f32)
    ss = lax.dot_general(oh, y, (((0,), (0,)), ((), ())),
                         preferred_element_type=_f32)
    cn = lax.dot_general(oh, jnp.ones((_BLK, 1), _f32),
                         (((0,), (0,)), ((), ())),
                         preferred_element_type=_f32)

    @pl.when(jnp.logical_and(c == 0, j == 0))
    def _():
        ss_ref[...] = jnp.zeros((2, G, D), _f32)
        cnt_ref[...] = jnp.zeros((2, G, 1), _f32)

    @pl.when(c == 0)
    def _():
        ss_ref[0] += ss
        cnt_ref[0] += cn

    @pl.when(c == 1)
    def _():
        ss_ref[1] += ss
        cnt_ref[1] += cn

    @pl.when(jnp.logical_and(c == 1, j == NP // _BLK - 1))
    def _():
        o = None
        for cidx in range(2):
            m = ss_ref[cidx] / jnp.maximum(cnt_ref[cidx], 1.0)
            z = jnp.dot(m, fcw_ref[cidx], preferred_element_type=_f32) \
                + fcb_ref[cidx]
            z = jnp.where(z >= 0, z, 0.01 * z)
            fw = fw_ref[pl.ds(cidx * D, D), :]
            contrib = jnp.dot(z, fw, preferred_element_type=_f32)
            o = contrib if o is None else o + contrib
        out_ref[...] = o + fb_ref[...]


def _pool_call(acc, dis_col, b_all, batch_all, fcw_all, fcb_all,
               final_w, final_b):
    full = lambda c, j: (0, 0)
    return pl.pallas_call(
        _pool_body,
        grid=(2, NP // _BLK),
        in_specs=[
            pl.BlockSpec((1, _BLK, D), lambda c, j: (c, j, 0)),
            pl.BlockSpec((1, _DROWS, 128), lambda c, j: (c, j, 0)),
            pl.BlockSpec((1, 1, D), lambda c, j: (c, 0, 0)),
            pl.BlockSpec((1, _DROWS, 128), lambda c, j: (c, j, 0)),
            pl.BlockSpec((2, D, D), lambda c, j: (0, 0, 0)),
            pl.BlockSpec((2, 1, D), lambda c, j: (0, 0, 0)),
            pl.BlockSpec((2 * D, 1), full),
            pl.BlockSpec((1, 1), full),
        ],
        out_specs=pl.BlockSpec((G, 1), full),
        out_shape=jax.ShapeDtypeStruct((G, 1), _f32),
        scratch_shapes=[
            pltpu.VMEM((2, G, D), _f32),
            pltpu.VMEM((2, G, 1), _f32),
        ],
    )(acc, dis_col, b_all, batch_all, fcw_all, fcb_all, final_w, final_b)


def kernel(pro1_x, pro1_edge_index, pro1_batch, pro2_x, pro2_edge_index,
           pro2_batch, W1, b1, fc1_w, fc1_b, W2, b2, fc2_w, fc2_b,
           final_w, final_b):
    e_all = jnp.stack([pro1_edge_index, pro2_edge_index])
    rowid = jnp.arange(DEG_ROWS, dtype=_i32)

    x_all = jnp.pad(jnp.stack([pro1_x, pro2_x]), ((0, 0), (0, NP - N), (0, 0)))
    w_all = jnp.stack([W1, W2])
    b_all = jnp.stack([b1, b2]).reshape(2, 1, D)
    batch_all = jnp.pad(jnp.stack([pro1_batch, pro2_batch]),
                        ((0, 0), (0, NP - N)),
                        constant_values=G).reshape(2, DEG_ROWS, 128)
    fcw_all = jnp.stack([fc1_w, fc2_w])
    fcb_all = jnp.stack([fc1_b, fc2_b]).reshape(2, 1, D)
    fb = final_b.reshape(1, 1)

    deg2d = _deg_call(e_all, rowid)
    g, dis_img = _g_call(x_all, deg2d, w_all)
    g_all = g.reshape(2 * NP, D)
    acc = _scatter_call(g_all, e_all)
    return _pool_call(acc, dis_img, b_all, batch_all, fcw_all, fcb_all,
                      final_w, fb)
